# Initial kernel scaffold; baseline (speedup 1.0000x reference)
import functools

import jax
import jax.numpy as jnp
from jax import lax
from jax.experimental import pallas as pl
from jax.experimental.pallas import tpu as pltpu
from jax.experimental.pallas import tpu_sc as plsc

EMBED_DIM = 32
NC, NS = 2, 16
NW = NC * NS
IDX_MINOR = 128
ROWS_PER_CHUNK = 1024
K = ROWS_PER_CHUNK // IDX_MINOR


def _body(n_chunks, idx_hbm, ktab_hbm, etab_hbm, kout_hbm, eout_hbm,
          idx_v, krows_v, erows_v, sem_k, sem_e):
    wid = lax.axis_index("s") * NC + lax.axis_index("c")
    base_row = wid * (n_chunks * ROWS_PER_CHUNK)

    def chunk_step(c, _):
        row0 = base_row + c * ROWS_PER_CHUNK
        irow0 = row0 // IDX_MINOR
        pltpu.sync_copy(idx_hbm.at[pl.ds(irow0, K)], idx_v)
        for j in range(K):
            pltpu.async_copy(
                ktab_hbm.at[idx_v.at[j]],
                krows_v.at[pl.ds(j * IDX_MINOR, IDX_MINOR)],
                sem_k,
            )
        for j in range(K):
            pltpu.async_copy(
                etab_hbm.at[idx_v.at[j]],
                erows_v.at[pl.ds(j * IDX_MINOR, IDX_MINOR)],
                sem_e,
            )
        pltpu.make_async_copy(
            ktab_hbm.at[pl.ds(0, ROWS_PER_CHUNK)], krows_v, sem_k
        ).wait()
        pltpu.make_async_copy(
            etab_hbm.at[pl.ds(0, ROWS_PER_CHUNK)], erows_v, sem_e
        ).wait()
        pltpu.sync_copy(krows_v, kout_hbm.at[pl.ds(row0, ROWS_PER_CHUNK)])
        pltpu.sync_copy(erows_v, eout_hbm.at[pl.ds(row0, ROWS_PER_CHUNK)])
        return ()

    lax.fori_loop(0, n_chunks, chunk_step, ())


def kernel(item_inputs, k_difficulty, e_discrimination):
    bsz, hist = item_inputs.shape
    b_total = bsz * hist
    assert b_total % (NW * ROWS_PER_CHUNK) == 0
    n_chunks = b_total // (NW * ROWS_PER_CHUNK)
    idx2d = item_inputs.reshape(b_total // IDX_MINOR, IDX_MINOR)

    mesh = plsc.VectorSubcoreMesh(core_axis_name="c", subcore_axis_name="s")
    run = pl.kernel(
        functools.partial(_body, n_chunks),
        out_type=(
            jax.ShapeDtypeStruct((b_total, EMBED_DIM), jnp.float32),
            jax.ShapeDtypeStruct((b_total, 1), jnp.float32),
        ),
        mesh=mesh,
        scratch_types=[
            pltpu.VMEM((K, IDX_MINOR), jnp.int32),
            pltpu.VMEM((ROWS_PER_CHUNK, EMBED_DIM), jnp.float32),
            pltpu.VMEM((ROWS_PER_CHUNK, 1), jnp.float32),
            pltpu.SemaphoreType.DMA,
            pltpu.SemaphoreType.DMA,
        ],
    )
    kout, eout = run(idx2d, k_difficulty, e_discrimination)
    return (kout.reshape(bsz, hist, EMBED_DIM), eout.reshape(bsz, hist, 1))


# speedup vs baseline: 24.3236x; 24.3236x over previous
import functools

import jax
import jax.numpy as jnp
from jax import lax
from jax.experimental import pallas as pl
from jax.experimental.pallas import tpu as pltpu
from jax.experimental.pallas import tpu_sc as plsc

EMBED_DIM = 32
NC, NS = 2, 16
NW = NC * NS
IDX_MINOR = 128
ROWS_PER_CHUNK = 1024
K = ROWS_PER_CHUNK // IDX_MINOR


def _body(n_chunks, idx_hbm, ktab_hbm, etab_hbm, kout_hbm, eout_hbm,
          idx_v, krows_v, evals_v, sem_k, sem_e):
    wid = lax.axis_index("s") * NC + lax.axis_index("c")
    base_row = wid * (n_chunks * ROWS_PER_CHUNK)

    def chunk_step(c, _):
        row0 = pl.multiple_of(base_row + c * ROWS_PER_CHUNK, ROWS_PER_CHUNK)
        irow0 = pl.multiple_of(row0 // IDX_MINOR, K)
        pltpu.sync_copy(idx_hbm.at[pl.ds(irow0, K)], idx_v)
        copies = []
        for j in range(K):
            copies.append(pltpu.async_copy(
                ktab_hbm.at[idx_v.at[j]],
                krows_v.at[pl.ds(j * IDX_MINOR, IDX_MINOR)],
                sem_k,
            ))
            copies.append(pltpu.async_copy(
                etab_hbm.at[idx_v.at[j]],
                evals_v.at[pl.ds(j * IDX_MINOR, IDX_MINOR)],
                sem_e,
            ))
        for cp in copies:
            cp.wait()
        pltpu.sync_copy(krows_v, kout_hbm.at[pl.ds(row0, ROWS_PER_CHUNK)])
        pltpu.sync_copy(evals_v, eout_hbm.at[pl.ds(row0, ROWS_PER_CHUNK)])
        return ()

    lax.fori_loop(0, n_chunks, chunk_step, ())


def kernel(item_inputs, k_difficulty, e_discrimination):
    bsz, hist = item_inputs.shape
    b_total = bsz * hist
    assert b_total % (NW * ROWS_PER_CHUNK) == 0
    n_chunks = b_total // (NW * ROWS_PER_CHUNK)
    idx2d = item_inputs.reshape(b_total // IDX_MINOR, IDX_MINOR)
    etab = e_discrimination.reshape(-1)

    mesh = plsc.VectorSubcoreMesh(core_axis_name="c", subcore_axis_name="s")
    run = pl.kernel(
        functools.partial(_body, n_chunks),
        out_type=(
            jax.ShapeDtypeStruct((b_total, EMBED_DIM), jnp.float32),
            jax.ShapeDtypeStruct((b_total,), jnp.float32),
        ),
        mesh=mesh,
        compiler_params=pltpu.CompilerParams(use_tc_tiling_on_sc=False),
        scratch_types=[
            pltpu.VMEM((K, IDX_MINOR), jnp.int32),
            pltpu.VMEM((ROWS_PER_CHUNK, EMBED_DIM), jnp.float32),
            pltpu.VMEM((ROWS_PER_CHUNK,), jnp.float32),
            pltpu.SemaphoreType.DMA,
            pltpu.SemaphoreType.DMA,
        ],
    )
    kout, eout = run(idx2d, k_difficulty, etab)
    return (kout.reshape(bsz, hist, EMBED_DIM), eout.reshape(bsz, hist, 1))


# speedup vs baseline: 25.4617x; 1.0468x over previous
import functools

import jax
import jax.numpy as jnp
from jax import lax
from jax.experimental import pallas as pl
from jax.experimental.pallas import tpu as pltpu
from jax.experimental.pallas import tpu_sc as plsc

EMBED_DIM = 32
NC, NS = 2, 16
NW = NC * NS
IDX_MINOR = 128
ROWS_PER_CHUNK = 1024
K = ROWS_PER_CHUNK // IDX_MINOR


def _body(n_chunks, idx_hbm, ktab_hbm, etab_hbm, kout_hbm, eout_hbm,
          idx_v, krows_v, evals_v,
          sem_i0, sem_i1, sem_k0, sem_k1, sem_e0, sem_e1, sem_w0, sem_w1):
    wid = lax.axis_index("s") * NC + lax.axis_index("c")
    base_row = wid * (n_chunks * ROWS_PER_CHUNK)
    sem_i = (sem_i0, sem_i1)
    sem_k = (sem_k0, sem_k1)
    sem_e = (sem_e0, sem_e1)
    sem_w = (sem_w0, sem_w1)

    def rows(c):
        return pl.multiple_of(base_row + c * ROWS_PER_CHUNK, ROWS_PER_CHUNK)

    def fire_idx(c, b):
        irow0 = pl.multiple_of(rows(c) // IDX_MINOR, K)
        pltpu.async_copy(idx_hbm.at[pl.ds(irow0, K)], idx_v.at[b], sem_i[b])

    def wait_idx(b):
        pltpu.make_async_copy(
            idx_hbm.at[pl.ds(0, K)], idx_v.at[b], sem_i[b]).wait()

    def fire_gathers(b):
        for j in range(K):
            pltpu.async_copy(
                ktab_hbm.at[idx_v.at[b].at[j]],
                krows_v.at[b].at[pl.ds(j * IDX_MINOR, IDX_MINOR)],
                sem_k[b],
            )
            pltpu.async_copy(
                etab_hbm.at[idx_v.at[b].at[j]],
                evals_v.at[b].at[pl.ds(j * IDX_MINOR, IDX_MINOR)],
                sem_e[b],
            )

    def wait_gathers(b):
        pltpu.make_async_copy(
            ktab_hbm.at[pl.ds(0, ROWS_PER_CHUNK)], krows_v.at[b],
            sem_k[b]).wait()
        pltpu.make_async_copy(
            etab_hbm.at[pl.ds(0, ROWS_PER_CHUNK)], evals_v.at[b],
            sem_e[b]).wait()

    def fire_wb(c, b):
        pltpu.async_copy(
            krows_v.at[b], kout_hbm.at[pl.ds(rows(c), ROWS_PER_CHUNK)],
            sem_w[b])
        pltpu.async_copy(
            evals_v.at[b], eout_hbm.at[pl.ds(rows(c), ROWS_PER_CHUNK)],
            sem_w[b])

    def wait_wb(b):
        pltpu.make_async_copy(
            krows_v.at[b], kout_hbm.at[pl.ds(0, ROWS_PER_CHUNK)],
            sem_w[b]).wait()
        pltpu.make_async_copy(
            evals_v.at[b], eout_hbm.at[pl.ds(0, ROWS_PER_CHUNK)],
            sem_w[b]).wait()

    fire_idx(0, 0)
    wait_idx(0)
    fire_gathers(0)
    fire_idx(1, 1)
    wait_idx(1)
    fire_gathers(1)
    wait_gathers(0)
    fire_wb(0, 0)
    fire_idx(2, 0)

    def step(h, _):
        c0 = 2 * h
        c1 = c0 + 1
        wait_gathers(1)
        fire_wb(c1 - 2, 1)
        fire_idx(c1, 1)
        wait_idx(0)
        wait_wb(0)
        fire_gathers(0)
        wait_gathers(0)
        fire_wb(c0, 0)
        fire_idx(jnp.minimum(c0 + 2, n_chunks - 1), 0)
        wait_idx(1)
        wait_wb(1)
        fire_gathers(1)
        return ()

    lax.fori_loop(1, n_chunks // 2, step, ())

    wait_gathers(1)
    fire_wb(n_chunks - 1, 1)
    wait_idx(0)
    wait_wb(0)
    wait_wb(1)


def kernel(item_inputs, k_difficulty, e_discrimination):
    bsz, hist = item_inputs.shape
    b_total = bsz * hist
    assert b_total % (NW * ROWS_PER_CHUNK) == 0
    n_chunks = b_total // (NW * ROWS_PER_CHUNK)
    assert n_chunks % 2 == 0 and n_chunks >= 4
    idx2d = item_inputs.reshape(b_total // IDX_MINOR, IDX_MINOR)
    etab = e_discrimination.reshape(-1)

    mesh = plsc.VectorSubcoreMesh(core_axis_name="c", subcore_axis_name="s")
    run = pl.kernel(
        functools.partial(_body, n_chunks),
        out_type=(
            jax.ShapeDtypeStruct((b_total, EMBED_DIM), jnp.float32),
            jax.ShapeDtypeStruct((b_total,), jnp.float32),
        ),
        mesh=mesh,
        compiler_params=pltpu.CompilerParams(use_tc_tiling_on_sc=False),
        scratch_types=[
            pltpu.VMEM((2, K, IDX_MINOR), jnp.int32),
            pltpu.VMEM((2, ROWS_PER_CHUNK, EMBED_DIM), jnp.float32),
            pltpu.VMEM((2, ROWS_PER_CHUNK), jnp.float32),
        ] + [pltpu.SemaphoreType.DMA] * 8,
    )
    kout, eout = run(idx2d, k_difficulty, etab)
    return (kout.reshape(bsz, hist, EMBED_DIM), eout.reshape(bsz, hist, 1))
